# Initial kernel scaffold; baseline (speedup 1.0000x reference)
import jax
import jax.numpy as jnp
from jax.experimental import pallas as pl


def kernel(x, edge_index, Wk1, Wq1, Wv1, a_rel1, m_rel1, pri1, Wa1, Wk2, Wq2, Wv2, a_rel2, m_rel2, pri2, Wa2):
    raise NotImplementedError("write your pallas kernel here")


# speedup vs baseline: 6.4086x; 6.4086x over previous
import functools

import jax
import jax.numpy as jnp
from jax import lax
from jax.experimental import pallas as pl
from jax.experimental.pallas import tpu as pltpu
from jax.experimental.pallas import tpu_sc as plsc

N = 10000
E = 320000
DIN = 128
HID = 512
H1 = 8
D1 = 64
OUT = 2

NC = 2
NS = 16
NW = NC * NS
EW = E // NW
CE = 80
NCH = EW // CE
NP = 10240
RPT = NP // NS
BN = 1000


def _iota16():
    return lax.iota(jnp.int32, 16)


def _proj1_body(x_ref, wk_ref, wq_ref, wv_ref, ar_ref, mr_ref, qs_ref,
                k_out, q_out, v_out):
    xb = x_ref[...]
    k = jnp.dot(xb, wk_ref[...], preferred_element_type=jnp.float32)
    q = jnp.dot(xb, wq_ref[...], preferred_element_type=jnp.float32)
    v = jnp.dot(xb, wv_ref[...], preferred_element_type=jnp.float32)
    q_out[...] = q * qs_ref[...]
    for h in range(H1):
        lo, hi = h * D1, (h + 1) * D1
        k_out[:, lo:hi] = jnp.dot(k[:, lo:hi], ar_ref[h], preferred_element_type=jnp.float32)
        v_out[:, lo:hi] = jnp.dot(v[:, lo:hi], mr_ref[h], preferred_element_type=jnp.float32)


def _proj1(x, wk, wq, wv, ar, mr, qscale):
    f = pl.pallas_call(
        _proj1_body,
        grid=(N // BN,),
        in_specs=[
            pl.BlockSpec((BN, DIN), lambda i: (i, 0)),
            pl.BlockSpec((DIN, HID), lambda i: (0, 0)),
            pl.BlockSpec((DIN, HID), lambda i: (0, 0)),
            pl.BlockSpec((DIN, HID), lambda i: (0, 0)),
            pl.BlockSpec((H1, D1, D1), lambda i: (0, 0, 0)),
            pl.BlockSpec((H1, D1, D1), lambda i: (0, 0, 0)),
            pl.BlockSpec((1, HID), lambda i: (0, 0)),
        ],
        out_specs=[
            pl.BlockSpec((BN, HID), lambda i: (i, 0)),
            pl.BlockSpec((BN, HID), lambda i: (i, 0)),
            pl.BlockSpec((BN, HID), lambda i: (i, 0)),
        ],
        out_shape=[jax.ShapeDtypeStruct((N, HID), jnp.float32)] * 3,
    )
    return f(x, wk, wq, wv, ar, mr, qscale)


def _attn1_body(k_hbm, q_hbm, src_hbm, dst_hbm, ex_hbm, den_hbm,
                ksrc, qdst, srcv, dstv, exrows, exbuf, zbuf, den_sh,
                sem1, sem2):
    c = lax.axis_index("c")
    s = lax.axis_index("s")
    wid = c * NS + s
    e0 = wid * EW

    def zrow(i, _):
        exrows[i, :] = jnp.zeros((16,), jnp.float32)
        return 0
    lax.fori_loop(0, CE, zrow, 0)

    def zline(i, _):
        zbuf[i, :] = jnp.zeros((16,), jnp.float32)
        return 0
    lax.fori_loop(0, 128, zline, 0)
    for j in range(5):
        pltpu.sync_copy(zbuf, den_sh.at[pl.ds(s * RPT + j * 128, 128)])
    plsc.subcore_barrier()

    def chunk(i, _):
        off = e0 + i * CE
        pltpu.sync_copy(src_hbm.at[pl.ds(off, CE)], srcv)
        pltpu.sync_copy(dst_hbm.at[pl.ds(off, CE)], dstv)
        cp1 = pltpu.async_copy(k_hbm.at[srcv], ksrc, sem1)
        cp2 = pltpu.async_copy(q_hbm.at[dstv], qdst, sem2)
        cp1.wait()
        cp2.wait()
        for g in range(CE // 16):
            rows = g * 16 + _iota16()

            def hbody(h, _):
                def fstep(t, acc):
                    base = h * D1 + t * 16
                    for j in range(16):
                        cv = jnp.full((16,), base + j, jnp.int32)
                        kv = plsc.load_gather(ksrc, [rows, cv])
                        qv = plsc.load_gather(qdst, [rows, cv])
                        acc = acc + kv * qv
                    return acc
                acc = lax.fori_loop(0, 4, fstep, jnp.zeros((16,), jnp.float32))
                ex = jnp.exp(acc)
                exbuf[h, pl.ds(g * 16, 16)] = ex
                plsc.store_scatter(exrows, [rows, jnp.full((16,), h, jnp.int32)], ex)
                return 0
            lax.fori_loop(0, H1, hbody, 0)
        for h in range(H1):
            pltpu.sync_copy(exbuf.at[h], ex_hbm.at[pl.ds(h * E + off, CE)])
        pltpu.sync_copy(exrows, den_sh.at[dstv], add=True)
        return 0
    lax.fori_loop(0, NCH, chunk, 0)

    plsc.subcore_barrier()
    pltpu.sync_copy(den_sh.at[pl.ds(s * RPT, RPT)],
                    den_hbm.at[c, pl.ds(s * RPT, RPT)])


def _attn1(k, q, src, dst):
    f = pl.kernel(
        _attn1_body,
        mesh=plsc.VectorSubcoreMesh(core_axis_name="c", subcore_axis_name="s"),
        compiler_params=pltpu.CompilerParams(use_tc_tiling_on_sc=False, needs_layout_passes=False),
        out_type=[
            jax.ShapeDtypeStruct((H1 * E,), jnp.float32),
            jax.ShapeDtypeStruct((NC, NP, 16), jnp.float32),
        ],
        scratch_types=[
            pltpu.VMEM((CE, HID), jnp.float32),
            pltpu.VMEM((CE, HID), jnp.float32),
            pltpu.VMEM((CE,), jnp.int32),
            pltpu.VMEM((CE,), jnp.int32),
            pltpu.VMEM((CE, 16), jnp.float32),
            pltpu.VMEM((H1, CE), jnp.float32),
            pltpu.VMEM((128, 16), jnp.float32),
            pltpu.VMEM_SHARED((NP, 16), jnp.float32),
            pltpu.SemaphoreType.DMA,
            pltpu.SemaphoreType.DMA,
        ],
    )
    return f(k, q, src, dst)


def _aggr1_body(v_hbm, src_hbm, dst_hbm, ex_hbm, agg_hbm,
                vrows, srcv, dstv, idxv, exv0, exv1, zbuf, agg_sh, sem):
    c = lax.axis_index("c")
    s = lax.axis_index("s")
    wid = c * NS + s
    e0 = wid * EW

    def zline(i, _):
        for j in range(8):
            zbuf[i, pl.ds(j * 16, 16)] = jnp.zeros((16,), jnp.float32)
        return 0
    lax.fori_loop(0, 128, zline, 0)

    def hpass(hp, _):
        for j in range(5):
            pltpu.sync_copy(zbuf, agg_sh.at[pl.ds(s * RPT + j * 128, 128)])
        plsc.subcore_barrier()

        def chunk(i, _):
            off = e0 + i * CE
            pltpu.sync_copy(src_hbm.at[pl.ds(off, CE)], srcv)
            pltpu.sync_copy(dst_hbm.at[pl.ds(off, CE)], dstv)
            for g in range(CE // 16):
                sl = pl.ds(g * 16, 16)
                idxv[sl] = srcv[sl] * 4 + hp
            pltpu.async_copy(v_hbm.at[idxv], vrows, sem).wait()
            pltpu.sync_copy(ex_hbm.at[pl.ds((2 * hp) * E + off, CE)], exv0)
            pltpu.sync_copy(ex_hbm.at[pl.ds((2 * hp + 1) * E + off, CE)], exv1)

            def sgrp(g, _):
                exg0 = exv0[pl.ds(g * 16, 16)]
                exg1 = exv1[pl.ds(g * 16, 16)]
                for j in range(16):
                    r = g * 16 + j
                    s0 = exg0[j]
                    s1 = exg1[j]
                    for t in range(4):
                        sl = pl.ds(t * 16, 16)
                        vrows[r, sl] = vrows[r, sl] * s0
                    for t in range(4):
                        sl = pl.ds(64 + t * 16, 16)
                        vrows[r, sl] = vrows[r, sl] * s1
                return 0
            lax.fori_loop(0, CE // 16, sgrp, 0)
            pltpu.sync_copy(vrows, agg_sh.at[dstv], add=True)
            return 0
        lax.fori_loop(0, NCH, chunk, 0)

        plsc.subcore_barrier()
        pltpu.sync_copy(agg_sh.at[pl.ds(s * RPT, RPT)],
                        agg_hbm.at[c, pl.ds(s * RPT, RPT), pl.ds(hp * 128, 128)])
        plsc.subcore_barrier()
        return 0
    lax.fori_loop(0, H1 // 2, hpass, 0)


def _aggr1(v4, src, dst, exh):
    f = pl.kernel(
        _aggr1_body,
        mesh=plsc.VectorSubcoreMesh(core_axis_name="c", subcore_axis_name="s"),
        compiler_params=pltpu.CompilerParams(use_tc_tiling_on_sc=False, needs_layout_passes=False),
        out_type=[jax.ShapeDtypeStruct((NC, NP, HID), jnp.float32)],
        scratch_types=[
            pltpu.VMEM((CE, 128), jnp.float32),
            pltpu.VMEM((CE,), jnp.int32),
            pltpu.VMEM((CE,), jnp.int32),
            pltpu.VMEM((CE,), jnp.int32),
            pltpu.VMEM((CE,), jnp.float32),
            pltpu.VMEM((CE,), jnp.float32),
            pltpu.VMEM((128, 128), jnp.float32),
            pltpu.VMEM_SHARED((NP, 128), jnp.float32),
            pltpu.SemaphoreType.DMA,
        ],
    )
    return f(v4, src, dst, exh)[0]


def _mid_body(ap_ref, dp_ref, wa_ref, wk2_ref, wq2_ref, wv2_ref,
              ar2_ref, mr2_ref, pr2_ref, kqv_out):
    agg = ap_ref[0] + ap_ref[1]
    den = dp_ref[0] + dp_ref[1]
    den8 = den[:, :H1] + 1e-16
    div = jnp.repeat(den8, D1, axis=1)
    h1 = jax.nn.gelu(agg / div)
    h1 = jnp.dot(h1, wa_ref[...], preferred_element_type=jnp.float32)
    wk2e = jnp.dot(wk2_ref[...], ar2_ref[...], preferred_element_type=jnp.float32)
    wv2e = jnp.dot(wv2_ref[...], mr2_ref[...], preferred_element_type=jnp.float32)
    k2 = jnp.dot(h1, wk2e, preferred_element_type=jnp.float32)
    q2 = jnp.dot(h1, wq2_ref[...], preferred_element_type=jnp.float32)
    q2 = q2 * (pr2_ref[0, 0] / jnp.sqrt(jnp.float32(OUT)))
    v2 = jnp.dot(h1, wv2e, preferred_element_type=jnp.float32)
    kqv_out[...] = jnp.concatenate(
        [k2, q2, v2, jnp.zeros((BN, 10), jnp.float32)], axis=1)


def _mid(agg1, den1, wa, wk2, wq2, wv2, ar2, mr2, pr2):
    f = pl.pallas_call(
        _mid_body,
        grid=(N // BN,),
        in_specs=[
            pl.BlockSpec((NC, BN, HID), lambda i: (0, i, 0)),
            pl.BlockSpec((NC, BN, 16), lambda i: (0, i, 0)),
            pl.BlockSpec((HID, HID), lambda i: (0, 0)),
            pl.BlockSpec((HID, OUT), lambda i: (0, 0)),
            pl.BlockSpec((HID, OUT), lambda i: (0, 0)),
            pl.BlockSpec((HID, OUT), lambda i: (0, 0)),
            pl.BlockSpec((OUT, OUT), lambda i: (0, 0)),
            pl.BlockSpec((OUT, OUT), lambda i: (0, 0)),
            pl.BlockSpec((1, 1), lambda i: (0, 0)),
        ],
        out_specs=[pl.BlockSpec((BN, 16), lambda i: (i, 0))],
        out_shape=[jax.ShapeDtypeStruct((N, 16), jnp.float32)],
    )
    return f(agg1, den1, wa, wk2, wq2, wv2, ar2, mr2, pr2)[0]


def _l2_body(kqv_hbm, src_hbm, dst_hbm, out_hbm,
             srows, drows, srcv, dstv, outrows, zbuf, out_sh, sem1, sem2):
    c = lax.axis_index("c")
    s = lax.axis_index("s")
    wid = c * NS + s
    e0 = wid * EW

    def zrow(i, _):
        outrows[i, :] = jnp.zeros((16,), jnp.float32)
        return 0
    lax.fori_loop(0, CE, zrow, 0)

    def zline(i, _):
        zbuf[i, :] = jnp.zeros((16,), jnp.float32)
        return 0
    lax.fori_loop(0, 128, zline, 0)
    for j in range(5):
        pltpu.sync_copy(zbuf, out_sh.at[pl.ds(s * RPT + j * 128, 128)])
    plsc.subcore_barrier()

    def chunk(i, _):
        off = e0 + i * CE
        pltpu.sync_copy(src_hbm.at[pl.ds(off, CE)], srcv)
        pltpu.sync_copy(dst_hbm.at[pl.ds(off, CE)], dstv)
        cp1 = pltpu.async_copy(kqv_hbm.at[srcv], srows, sem1)
        cp2 = pltpu.async_copy(kqv_hbm.at[dstv], drows, sem2)
        cp1.wait()
        cp2.wait()
        for g in range(CE // 16):
            rows = g * 16 + _iota16()
            k0 = plsc.load_gather(srows, [rows, jnp.full((16,), 0, jnp.int32)])
            k1 = plsc.load_gather(srows, [rows, jnp.full((16,), 1, jnp.int32)])
            q0 = plsc.load_gather(drows, [rows, jnp.full((16,), 2, jnp.int32)])
            q1 = plsc.load_gather(drows, [rows, jnp.full((16,), 3, jnp.int32)])
            v0 = plsc.load_gather(srows, [rows, jnp.full((16,), 4, jnp.int32)])
            v1 = plsc.load_gather(srows, [rows, jnp.full((16,), 5, jnp.int32)])
            ex = jnp.exp(k0 * q0 + k1 * q1)
            plsc.store_scatter(outrows, [rows, jnp.full((16,), 0, jnp.int32)], v0 * ex)
            plsc.store_scatter(outrows, [rows, jnp.full((16,), 1, jnp.int32)], v1 * ex)
            plsc.store_scatter(outrows, [rows, jnp.full((16,), 2, jnp.int32)], ex)
        pltpu.sync_copy(outrows, out_sh.at[dstv], add=True)
        return 0
    lax.fori_loop(0, NCH, chunk, 0)

    plsc.subcore_barrier()
    pltpu.sync_copy(out_sh.at[pl.ds(s * RPT, RPT)],
                    out_hbm.at[c, pl.ds(s * RPT, RPT)])


def _l2(kqv2, src, dst):
    f = pl.kernel(
        _l2_body,
        mesh=plsc.VectorSubcoreMesh(core_axis_name="c", subcore_axis_name="s"),
        compiler_params=pltpu.CompilerParams(use_tc_tiling_on_sc=False, needs_layout_passes=False),
        out_type=[jax.ShapeDtypeStruct((NC, NP, 16), jnp.float32)],
        scratch_types=[
            pltpu.VMEM((CE, 16), jnp.float32),
            pltpu.VMEM((CE, 16), jnp.float32),
            pltpu.VMEM((CE,), jnp.int32),
            pltpu.VMEM((CE,), jnp.int32),
            pltpu.VMEM((CE, 16), jnp.float32),
            pltpu.VMEM((128, 16), jnp.float32),
            pltpu.VMEM_SHARED((NP, 16), jnp.float32),
            pltpu.SemaphoreType.DMA,
            pltpu.SemaphoreType.DMA,
        ],
    )
    return f(kqv2, src, dst)[0]


def _out_body(p2_ref, wa2_ref, o_ref):
    t = p2_ref[0] + p2_ref[1]
    num = t[:N, 0:2]
    den = t[:N, 2:3] + 1e-16
    o_ref[...] = jnp.dot(jax.nn.gelu(num / den), wa2_ref[...],
                         preferred_element_type=jnp.float32)


def _out(p2, wa2):
    f = pl.pallas_call(
        _out_body,
        in_specs=[
            pl.BlockSpec((NC, NP, 16), lambda: (0, 0, 0)),
            pl.BlockSpec((OUT, OUT), lambda: (0, 0)),
        ],
        out_specs=[pl.BlockSpec((N, OUT), lambda: (0, 0))],
        out_shape=[jax.ShapeDtypeStruct((N, OUT), jnp.float32)],
    )
    return f(p2, wa2)[0]


def kernel(x, edge_index, Wk1, Wq1, Wv1, a_rel1, m_rel1, pri1, Wa1,
           Wk2, Wq2, Wv2, a_rel2, m_rel2, pri2, Wa2):
    src = edge_index[0].astype(jnp.int32)
    dst = edge_index[1].astype(jnp.int32)
    qscale = (jnp.repeat(pri1, D1) / jnp.sqrt(jnp.float32(D1))).reshape(1, HID)
    k1, q1, v1 = _proj1(x, Wk1, Wq1, Wv1, a_rel1, m_rel1, qscale)
    exh, den1 = _attn1(k1, q1, src, dst)
    agg1 = _aggr1(v1.reshape(N * 4, 128), src, dst, exh)
    kqv2 = _mid(agg1, den1, Wa1, Wk2, Wq2, Wv2,
                a_rel2.reshape(OUT, OUT), m_rel2.reshape(OUT, OUT),
                pri2.reshape(1, 1))
    p2 = _l2(kqv2, src, dst)
    return _out(p2, Wa2)


# speedup vs baseline: 13.2401x; 2.0660x over previous
import functools

import jax
import jax.numpy as jnp
from jax import lax
from jax.experimental import pallas as pl
from jax.experimental.pallas import tpu as pltpu
from jax.experimental.pallas import tpu_sc as plsc

N = 10000
E = 320000
DIN = 128
HID = 512
H1 = 8
D1 = 64
OUT = 2

NC = 2
NS = 16
NW = NC * NS
EW = E // NW
CE = 80
NCH = EW // CE
NP = 10240
RPT = NP // NS
BN = 1000


def _iota16():
    return lax.iota(jnp.int32, 16)


def _perm_cols(w):
    w3 = w.reshape(w.shape[0], H1, D1 // 2, 2)
    e = w3[:, :, :, 0]
    o = w3[:, :, :, 1]
    o = jnp.concatenate([o[:, H1 - 1 - i:H1 - i] for i in range(H1)], axis=1)
    t = jnp.concatenate([e.transpose(0, 2, 1), o.transpose(0, 2, 1)], axis=2)
    return t.reshape(w.shape[0], HID)


def _proj1_body(x_ref, wk_ref, wq_ref, wv_ref, ar_ref, mr_ref, qs_ref,
                k_out, q_out, v_out):
    xb = x_ref[...]
    wk = wk_ref[...]
    ke = jnp.concatenate(
        [jnp.dot(wk[:, h * D1:(h + 1) * D1], ar_ref[h],
                 preferred_element_type=jnp.float32) for h in range(H1)],
        axis=1)
    qe = wq_ref[...] * qs_ref[...]
    k_out[...] = jnp.dot(xb, _perm_cols(ke), preferred_element_type=jnp.float32)
    q_out[...] = jnp.dot(xb, _perm_cols(qe), preferred_element_type=jnp.float32)
    v = jnp.dot(xb, wv_ref[...], preferred_element_type=jnp.float32)
    for h in range(H1):
        lo, hi = h * D1, (h + 1) * D1
        v_out[:, lo:hi] = jnp.dot(v[:, lo:hi], mr_ref[h], preferred_element_type=jnp.float32)


def _proj1(x, wk, wq, wv, ar, mr, qscale):
    f = pl.pallas_call(
        _proj1_body,
        grid=(N // BN,),
        in_specs=[
            pl.BlockSpec((BN, DIN), lambda i: (i, 0)),
            pl.BlockSpec((DIN, HID), lambda i: (0, 0)),
            pl.BlockSpec((DIN, HID), lambda i: (0, 0)),
            pl.BlockSpec((DIN, HID), lambda i: (0, 0)),
            pl.BlockSpec((H1, D1, D1), lambda i: (0, 0, 0)),
            pl.BlockSpec((H1, D1, D1), lambda i: (0, 0, 0)),
            pl.BlockSpec((1, HID), lambda i: (0, 0)),
        ],
        out_specs=[
            pl.BlockSpec((BN, HID), lambda i: (i, 0)),
            pl.BlockSpec((BN, HID), lambda i: (i, 0)),
            pl.BlockSpec((BN, HID), lambda i: (i, 0)),
        ],
        out_shape=[jax.ShapeDtypeStruct((N, HID), jnp.float32)] * 3,
    )
    return f(x, wk, wq, wv, ar, mr, qscale)


def _attn1_body(k_hbm, q_hbm, src_hbm, dst_hbm, ex_hbm, den_hbm,
                ksrc, qdst, srcv, dstv, exrows, exbuf, zbuf, den_sh,
                sem1, sem2):
    c = lax.axis_index("c")
    s = lax.axis_index("s")
    wid = c * NS + s
    e0 = wid * EW

    def zrow(i, _):
        exrows[i, :] = jnp.zeros((16,), jnp.float32)
        return 0
    lax.fori_loop(0, CE, zrow, 0)

    def zline(i, _):
        zbuf[i, :] = jnp.zeros((16,), jnp.float32)
        return 0
    lax.fori_loop(0, 128, zline, 0)
    for j in range(5):
        pltpu.sync_copy(zbuf, den_sh.at[pl.ds(s * RPT + j * 128, 128)])
    plsc.subcore_barrier()

    def chunk(i, _):
        off = e0 + i * CE
        pltpu.sync_copy(src_hbm.at[pl.ds(off, CE)], srcv)
        pltpu.sync_copy(dst_hbm.at[pl.ds(off, CE)], dstv)
        cp1 = pltpu.async_copy(k_hbm.at[srcv], ksrc, sem1)
        cp2 = pltpu.async_copy(q_hbm.at[dstv], qdst, sem2)
        cp1.wait()
        cp2.wait()
        m8 = _iota16() < 8

        def ebody(r, _):
            accs = [jnp.zeros((16,), jnp.float32) for _ in range(4)]
            for t in range(HID // 16):
                sl = pl.ds(t * 16, 16)
                accs[t % 4] = accs[t % 4] + ksrc[r, sl] * qdst[r, sl]
            tot = (accs[0] + accs[1]) + (accs[2] + accs[3])
            tot = tot + lax.rev(tot, (0,))
            ex = jnp.exp(tot)
            rv = jnp.full((16,), r, jnp.int32)
            plsc.store_scatter(exrows, [rv, _iota16()], ex, mask=m8)
            plsc.store_scatter(exbuf, [_iota16(), rv], ex, mask=m8)
            return 0
        lax.fori_loop(0, CE, ebody, 0)
        for h in range(H1):
            pltpu.sync_copy(exbuf.at[h, pl.ds(0, CE)], ex_hbm.at[pl.ds(h * E + off, CE)])
        pltpu.sync_copy(exrows, den_sh.at[dstv], add=True)
        return 0
    lax.fori_loop(0, NCH, chunk, 0)

    plsc.subcore_barrier()
    pltpu.sync_copy(den_sh.at[pl.ds(s * RPT, RPT)],
                    den_hbm.at[c, pl.ds(s * RPT, RPT)])


def _attn1(k, q, src, dst):
    f = pl.kernel(
        _attn1_body,
        mesh=plsc.VectorSubcoreMesh(core_axis_name="c", subcore_axis_name="s"),
        compiler_params=pltpu.CompilerParams(use_tc_tiling_on_sc=False, needs_layout_passes=False),
        out_type=[
            jax.ShapeDtypeStruct((H1 * E,), jnp.float32),
            jax.ShapeDtypeStruct((NC, NP, 16), jnp.float32),
        ],
        scratch_types=[
            pltpu.VMEM((CE, HID), jnp.float32),
            pltpu.VMEM((CE, HID), jnp.float32),
            pltpu.VMEM((CE,), jnp.int32),
            pltpu.VMEM((CE,), jnp.int32),
            pltpu.VMEM((CE, 16), jnp.float32),
            pltpu.VMEM((H1, CE + 1), jnp.float32),
            pltpu.VMEM((128, 16), jnp.float32),
            pltpu.VMEM_SHARED((NP, 16), jnp.float32),
            pltpu.SemaphoreType.DMA,
            pltpu.SemaphoreType.DMA,
        ],
    )
    return f(k, q, src, dst)


def _aggr1_body(v_hbm, src_hbm, dst_hbm, ex_hbm, agg_hbm,
                vrows, srcv, dstv, idxv, exv0, exv1, zbuf, agg_sh, sem):
    c = lax.axis_index("c")
    s = lax.axis_index("s")
    wid = c * NS + s
    e0 = wid * EW

    def zline(i, _):
        for j in range(8):
            zbuf[i, pl.ds(j * 16, 16)] = jnp.zeros((16,), jnp.float32)
        return 0
    lax.fori_loop(0, 128, zline, 0)

    def hpass(hp, _):
        for j in range(5):
            pltpu.sync_copy(zbuf, agg_sh.at[pl.ds(s * RPT + j * 128, 128)])
        plsc.subcore_barrier()

        def chunk(i, _):
            off = e0 + i * CE
            pltpu.sync_copy(src_hbm.at[pl.ds(off, CE)], srcv)
            pltpu.sync_copy(dst_hbm.at[pl.ds(off, CE)], dstv)
            for g in range(CE // 16):
                sl = pl.ds(g * 16, 16)
                idxv[sl] = srcv[sl] * 4 + hp
            pltpu.async_copy(v_hbm.at[idxv], vrows, sem).wait()
            pltpu.sync_copy(ex_hbm.at[pl.ds((2 * hp) * E + off, CE)], exv0)
            pltpu.sync_copy(ex_hbm.at[pl.ds((2 * hp + 1) * E + off, CE)], exv1)

            def sgrp(g, _):
                exg0 = exv0[pl.ds(g * 16, 16)]
                exg1 = exv1[pl.ds(g * 16, 16)]
                for j in range(16):
                    r = g * 16 + j
                    s0 = exg0[j]
                    s1 = exg1[j]
                    for t in range(4):
                        sl = pl.ds(t * 16, 16)
                        vrows[r, sl] = vrows[r, sl] * s0
                    for t in range(4):
                        sl = pl.ds(64 + t * 16, 16)
                        vrows[r, sl] = vrows[r, sl] * s1
                return 0
            lax.fori_loop(0, CE // 16, sgrp, 0)
            pltpu.sync_copy(vrows, agg_sh.at[dstv], add=True)
            return 0
        lax.fori_loop(0, NCH, chunk, 0)

        plsc.subcore_barrier()
        pltpu.sync_copy(agg_sh.at[pl.ds(s * RPT, RPT)],
                        agg_hbm.at[c, pl.ds(s * RPT, RPT), pl.ds(hp * 128, 128)])
        plsc.subcore_barrier()
        return 0
    lax.fori_loop(0, H1 // 2, hpass, 0)


def _aggr1(v4, src, dst, exh):
    f = pl.kernel(
        _aggr1_body,
        mesh=plsc.VectorSubcoreMesh(core_axis_name="c", subcore_axis_name="s"),
        compiler_params=pltpu.CompilerParams(use_tc_tiling_on_sc=False, needs_layout_passes=False),
        out_type=[jax.ShapeDtypeStruct((NC, NP, HID), jnp.float32)],
        scratch_types=[
            pltpu.VMEM((CE, 128), jnp.float32),
            pltpu.VMEM((CE,), jnp.int32),
            pltpu.VMEM((CE,), jnp.int32),
            pltpu.VMEM((CE,), jnp.int32),
            pltpu.VMEM((CE,), jnp.float32),
            pltpu.VMEM((CE,), jnp.float32),
            pltpu.VMEM((128, 128), jnp.float32),
            pltpu.VMEM_SHARED((NP, 128), jnp.float32),
            pltpu.SemaphoreType.DMA,
        ],
    )
    return f(v4, src, dst, exh)[0]


def _mid_body(ap_ref, dp_ref, wa_ref, wk2_ref, wq2_ref, wv2_ref,
              ar2_ref, mr2_ref, pr2_ref, kqv_out):
    agg = ap_ref[0] + ap_ref[1]
    den = dp_ref[0] + dp_ref[1]
    den8 = den[:, :H1] + 1e-16
    div = jnp.repeat(den8, D1, axis=1)
    h1 = jax.nn.gelu(agg / div)
    h1 = jnp.dot(h1, wa_ref[...], preferred_element_type=jnp.float32)
    wk2e = jnp.dot(wk2_ref[...], ar2_ref[...], preferred_element_type=jnp.float32)
    wv2e = jnp.dot(wv2_ref[...], mr2_ref[...], preferred_element_type=jnp.float32)
    k2 = jnp.dot(h1, wk2e, preferred_element_type=jnp.float32)
    q2 = jnp.dot(h1, wq2_ref[...], preferred_element_type=jnp.float32)
    q2 = q2 * (pr2_ref[0, 0] / jnp.sqrt(jnp.float32(OUT)))
    v2 = jnp.dot(h1, wv2e, preferred_element_type=jnp.float32)
    kqv_out[...] = jnp.concatenate(
        [k2, q2, v2, jnp.zeros((BN, 10), jnp.float32)], axis=1)


def _mid(agg1, den1, wa, wk2, wq2, wv2, ar2, mr2, pr2):
    f = pl.pallas_call(
        _mid_body,
        grid=(N // BN,),
        in_specs=[
            pl.BlockSpec((NC, BN, HID), lambda i: (0, i, 0)),
            pl.BlockSpec((NC, BN, 16), lambda i: (0, i, 0)),
            pl.BlockSpec((HID, HID), lambda i: (0, 0)),
            pl.BlockSpec((HID, OUT), lambda i: (0, 0)),
            pl.BlockSpec((HID, OUT), lambda i: (0, 0)),
            pl.BlockSpec((HID, OUT), lambda i: (0, 0)),
            pl.BlockSpec((OUT, OUT), lambda i: (0, 0)),
            pl.BlockSpec((OUT, OUT), lambda i: (0, 0)),
            pl.BlockSpec((1, 1), lambda i: (0, 0)),
        ],
        out_specs=[pl.BlockSpec((BN, 16), lambda i: (i, 0))],
        out_shape=[jax.ShapeDtypeStruct((N, 16), jnp.float32)],
    )
    return f(agg1, den1, wa, wk2, wq2, wv2, ar2, mr2, pr2)[0]


def _l2_body(kqv_hbm, src_hbm, dst_hbm, out_hbm,
             srows, drows, srcv, dstv, outrows, zbuf, out_sh, sem1, sem2):
    c = lax.axis_index("c")
    s = lax.axis_index("s")
    wid = c * NS + s
    e0 = wid * EW

    def zrow(i, _):
        outrows[i, :] = jnp.zeros((16,), jnp.float32)
        return 0
    lax.fori_loop(0, CE, zrow, 0)

    def zline(i, _):
        zbuf[i, :] = jnp.zeros((16,), jnp.float32)
        return 0
    lax.fori_loop(0, 128, zline, 0)
    for j in range(5):
        pltpu.sync_copy(zbuf, out_sh.at[pl.ds(s * RPT + j * 128, 128)])
    plsc.subcore_barrier()

    def chunk(i, _):
        off = e0 + i * CE
        pltpu.sync_copy(src_hbm.at[pl.ds(off, CE)], srcv)
        pltpu.sync_copy(dst_hbm.at[pl.ds(off, CE)], dstv)
        cp1 = pltpu.async_copy(kqv_hbm.at[srcv], srows, sem1)
        cp2 = pltpu.async_copy(kqv_hbm.at[dstv], drows, sem2)
        cp1.wait()
        cp2.wait()
        for g in range(CE // 16):
            rows = g * 16 + _iota16()
            k0 = plsc.load_gather(srows, [rows, jnp.full((16,), 0, jnp.int32)])
            k1 = plsc.load_gather(srows, [rows, jnp.full((16,), 1, jnp.int32)])
            q0 = plsc.load_gather(drows, [rows, jnp.full((16,), 2, jnp.int32)])
            q1 = plsc.load_gather(drows, [rows, jnp.full((16,), 3, jnp.int32)])
            v0 = plsc.load_gather(srows, [rows, jnp.full((16,), 4, jnp.int32)])
            v1 = plsc.load_gather(srows, [rows, jnp.full((16,), 5, jnp.int32)])
            ex = jnp.exp(k0 * q0 + k1 * q1)
            plsc.store_scatter(outrows, [rows, jnp.full((16,), 0, jnp.int32)], v0 * ex)
            plsc.store_scatter(outrows, [rows, jnp.full((16,), 1, jnp.int32)], v1 * ex)
            plsc.store_scatter(outrows, [rows, jnp.full((16,), 2, jnp.int32)], ex)
        pltpu.sync_copy(outrows, out_sh.at[dstv], add=True)
        return 0
    lax.fori_loop(0, NCH, chunk, 0)

    plsc.subcore_barrier()
    pltpu.sync_copy(out_sh.at[pl.ds(s * RPT, RPT)],
                    out_hbm.at[c, pl.ds(s * RPT, RPT)])


def _l2(kqv2, src, dst):
    f = pl.kernel(
        _l2_body,
        mesh=plsc.VectorSubcoreMesh(core_axis_name="c", subcore_axis_name="s"),
        compiler_params=pltpu.CompilerParams(use_tc_tiling_on_sc=False, needs_layout_passes=False),
        out_type=[jax.ShapeDtypeStruct((NC, NP, 16), jnp.float32)],
        scratch_types=[
            pltpu.VMEM((CE, 16), jnp.float32),
            pltpu.VMEM((CE, 16), jnp.float32),
            pltpu.VMEM((CE,), jnp.int32),
            pltpu.VMEM((CE,), jnp.int32),
            pltpu.VMEM((CE, 16), jnp.float32),
            pltpu.VMEM((128, 16), jnp.float32),
            pltpu.VMEM_SHARED((NP, 16), jnp.float32),
            pltpu.SemaphoreType.DMA,
            pltpu.SemaphoreType.DMA,
        ],
    )
    return f(kqv2, src, dst)[0]


def _out_body(p2_ref, wa2_ref, o_ref):
    t = p2_ref[0] + p2_ref[1]
    num = t[:N, 0:2]
    den = t[:N, 2:3] + 1e-16
    o_ref[...] = jnp.dot(jax.nn.gelu(num / den), wa2_ref[...],
                         preferred_element_type=jnp.float32)


def _out(p2, wa2):
    f = pl.pallas_call(
        _out_body,
        in_specs=[
            pl.BlockSpec((NC, NP, 16), lambda: (0, 0, 0)),
            pl.BlockSpec((OUT, OUT), lambda: (0, 0)),
        ],
        out_specs=[pl.BlockSpec((N, OUT), lambda: (0, 0))],
        out_shape=[jax.ShapeDtypeStruct((N, OUT), jnp.float32)],
    )
    return f(p2, wa2)[0]


def kernel(x, edge_index, Wk1, Wq1, Wv1, a_rel1, m_rel1, pri1, Wa1,
           Wk2, Wq2, Wv2, a_rel2, m_rel2, pri2, Wa2):
    src = edge_index[0].astype(jnp.int32)
    dst = edge_index[1].astype(jnp.int32)
    qscale = (jnp.repeat(pri1, D1) / jnp.sqrt(jnp.float32(D1))).reshape(1, HID)
    k1, q1, v1 = _proj1(x, Wk1, Wq1, Wv1, a_rel1, m_rel1, qscale)
    exh, den1 = _attn1(k1, q1, src, dst)
    agg1 = _aggr1(v1.reshape(N * 4, 128), src, dst, exh)
    kqv2 = _mid(agg1, den1, Wa1, Wk2, Wq2, Wv2,
                a_rel2.reshape(OUT, OUT), m_rel2.reshape(OUT, OUT),
                pri2.reshape(1, 1))
    p2 = _l2(kqv2, src, dst)
    return _out(p2, Wa2)


# speedup vs baseline: 15.9773x; 1.2067x over previous
import functools

import jax
import jax.numpy as jnp
from jax import lax
from jax.experimental import pallas as pl
from jax.experimental.pallas import tpu as pltpu
from jax.experimental.pallas import tpu_sc as plsc

N = 10000
E = 320000
DIN = 128
HID = 512
H1 = 8
D1 = 64
OUT = 2

NC = 2
NS = 16
NW = NC * NS
EW = E // NW
CE = 80
NCH = EW // CE
NP = 10240
RPT = NP // NS
BN = 1000


def _iota16():
    return lax.iota(jnp.int32, 16)


def _perm_cols(w):
    w3 = w.reshape(w.shape[0], H1, D1 // 2, 2)
    e = w3[:, :, :, 0]
    o = w3[:, :, :, 1]
    o = jnp.concatenate([o[:, H1 - 1 - i:H1 - i] for i in range(H1)], axis=1)
    t = jnp.concatenate([e.transpose(0, 2, 1), o.transpose(0, 2, 1)], axis=2)
    return t.reshape(w.shape[0], HID)


def _proj1_body(x_ref, wk_ref, wq_ref, wv_ref, ar_ref, mr_ref, qs_ref,
                k_out, q_out, v_out):
    xb = x_ref[...]
    wk = wk_ref[...]
    ke = jnp.concatenate(
        [jnp.dot(wk[:, h * D1:(h + 1) * D1], ar_ref[h],
                 preferred_element_type=jnp.float32) for h in range(H1)],
        axis=1)
    qe = wq_ref[...] * qs_ref[...]
    k_out[...] = jnp.dot(xb, _perm_cols(ke), preferred_element_type=jnp.float32)
    q_out[...] = jnp.dot(xb, _perm_cols(qe), preferred_element_type=jnp.float32)
    v = jnp.dot(xb, wv_ref[...], preferred_element_type=jnp.float32)
    for h in range(H1):
        lo, hi = h * D1, (h + 1) * D1
        v_out[:, lo:hi] = jnp.dot(v[:, lo:hi], mr_ref[h], preferred_element_type=jnp.float32)


def _proj1(x, wk, wq, wv, ar, mr, qscale):
    f = pl.pallas_call(
        _proj1_body,
        grid=(N // BN,),
        in_specs=[
            pl.BlockSpec((BN, DIN), lambda i: (i, 0)),
            pl.BlockSpec((DIN, HID), lambda i: (0, 0)),
            pl.BlockSpec((DIN, HID), lambda i: (0, 0)),
            pl.BlockSpec((DIN, HID), lambda i: (0, 0)),
            pl.BlockSpec((H1, D1, D1), lambda i: (0, 0, 0)),
            pl.BlockSpec((H1, D1, D1), lambda i: (0, 0, 0)),
            pl.BlockSpec((1, HID), lambda i: (0, 0)),
        ],
        out_specs=[
            pl.BlockSpec((BN, HID), lambda i: (i, 0)),
            pl.BlockSpec((BN, HID), lambda i: (i, 0)),
            pl.BlockSpec((BN, HID), lambda i: (i, 0)),
        ],
        out_shape=[jax.ShapeDtypeStruct((N, HID), jnp.float32)] * 3,
    )
    return f(x, wk, wq, wv, ar, mr, qscale)


def _attn1_body(k_hbm, q_hbm, src_hbm, dst_hbm, ex_hbm, den_hbm,
                ksrc, qdst, srcv, dstv, exrows, exbuf, zbuf, den_sh,
                sem1, sem2):
    c = lax.axis_index("c")
    s = lax.axis_index("s")
    wid = c * NS + s
    e0 = wid * EW

    def zrow(i, _):
        exrows[i, :] = jnp.zeros((16,), jnp.float32)
        return 0
    lax.fori_loop(0, CE, zrow, 0)

    def zline(i, _):
        zbuf[i, :] = jnp.zeros((16,), jnp.float32)
        return 0
    lax.fori_loop(0, 128, zline, 0)
    for j in range(5):
        pltpu.sync_copy(zbuf, den_sh.at[pl.ds(s * RPT + j * 128, 128)])
    plsc.subcore_barrier()

    def chunk(i, _):
        off = e0 + i * CE
        pltpu.sync_copy(src_hbm.at[pl.ds(off, CE)], srcv)
        pltpu.sync_copy(dst_hbm.at[pl.ds(off, CE)], dstv)
        cp1 = pltpu.async_copy(k_hbm.at[srcv], ksrc, sem1)
        cp2 = pltpu.async_copy(q_hbm.at[dstv], qdst, sem2)
        cp1.wait()
        cp2.wait()
        m8 = _iota16() < 8

        def ebody(r, _):
            accs = [jnp.zeros((16,), jnp.float32) for _ in range(4)]
            for t in range(HID // 16):
                sl = pl.ds(t * 16, 16)
                accs[t % 4] = accs[t % 4] + ksrc[r, sl] * qdst[r, sl]
            tot = (accs[0] + accs[1]) + (accs[2] + accs[3])
            tot = tot + lax.rev(tot, (0,))
            ex = jnp.exp(tot)
            rv = jnp.full((16,), r, jnp.int32)
            plsc.store_scatter(exrows, [rv, _iota16()], ex, mask=m8)
            plsc.store_scatter(exbuf, [_iota16(), rv], ex, mask=m8)
            return 0
        lax.fori_loop(0, CE, ebody, 0)
        for h in range(H1):
            pltpu.sync_copy(exbuf.at[h, pl.ds(0, CE)], ex_hbm.at[pl.ds(h * E + off, CE)])
        pltpu.sync_copy(exrows, den_sh.at[dstv], add=True)
        return 0
    lax.fori_loop(0, NCH, chunk, 0)

    plsc.subcore_barrier()
    pltpu.sync_copy(den_sh.at[pl.ds(s * RPT, RPT)],
                    den_hbm.at[c, pl.ds(s * RPT, RPT)])


def _attn1(k, q, src, dst):
    f = pl.kernel(
        _attn1_body,
        mesh=plsc.VectorSubcoreMesh(core_axis_name="c", subcore_axis_name="s"),
        compiler_params=pltpu.CompilerParams(use_tc_tiling_on_sc=False, needs_layout_passes=False),
        out_type=[
            jax.ShapeDtypeStruct((H1 * E,), jnp.float32),
            jax.ShapeDtypeStruct((NC, NP, 16), jnp.float32),
        ],
        scratch_types=[
            pltpu.VMEM((CE, HID), jnp.float32),
            pltpu.VMEM((CE, HID), jnp.float32),
            pltpu.VMEM((CE,), jnp.int32),
            pltpu.VMEM((CE,), jnp.int32),
            pltpu.VMEM((CE, 16), jnp.float32),
            pltpu.VMEM((H1, CE + 1), jnp.float32),
            pltpu.VMEM((128, 16), jnp.float32),
            pltpu.VMEM_SHARED((NP, 16), jnp.float32),
            pltpu.SemaphoreType.DMA,
            pltpu.SemaphoreType.DMA,
        ],
    )
    return f(k, q, src, dst)


def _aggr1_body(v_hbm, src_hbm, dst_hbm, ex_hbm, agg_hbm,
                grows0, grows1, srows0, srows1,
                srcv0, srcv1, dstv0, dstv1, idxv0, idxv1, dsca0, dsca1,
                exv0, exv1, zbuf, agg_sh, semg0, semg1, sems0, sems1):
    c = lax.axis_index("c")
    s = lax.axis_index("s")
    wid = c * NS + s
    e0 = wid * EW
    grows = (grows0, grows1)
    srows = (srows0, srows1)
    srcv = (srcv0, srcv1)
    dstv = (dstv0, dstv1)
    idxv = (idxv0, idxv1)
    dsca = (dsca0, dsca1)
    semg = (semg0, semg1)
    sems = (sems0, sems1)

    def zline(i, _):
        for j in range(8):
            zbuf[i, pl.ds(j * 16, 16)] = jnp.zeros((16,), jnp.float32)
        return 0
    lax.fori_loop(0, 16, zline, 0)

    def load_idx(ch, b, hp):
        off = e0 + (ch % NCH) * CE
        pltpu.sync_copy(src_hbm.at[pl.ds(off, CE)], srcv[b])
        pltpu.sync_copy(dst_hbm.at[pl.ds(off, CE)], dstv[b])
        for g in range(CE // 16):
            sl = pl.ds(g * 16, 16)
            idxv[b][sl] = srcv[b][sl] * 4 + hp
        pltpu.async_copy(v_hbm.at[idxv[b]], grows[b], semg[b])

    def process(ch, b, hp, wait_scatter):
        off = e0 + ch * CE
        pltpu.sync_copy(ex_hbm.at[pl.ds((2 * hp) * E + off, CE)], exv0)
        pltpu.sync_copy(ex_hbm.at[pl.ds((2 * hp + 1) * E + off, CE)], exv1)
        pltpu.make_async_copy(v_hbm.at[pl.ds(0, CE)], grows[b], semg[b]).wait()
        if wait_scatter is True:
            pltpu.make_async_copy(v_hbm.at[pl.ds(0, CE)], srows[b], sems[b]).wait()
        elif wait_scatter is not False:
            @pl.when(wait_scatter)
            def _():
                pltpu.make_async_copy(v_hbm.at[pl.ds(0, CE)], srows[b], sems[b]).wait()

        def sgrp(g, _):
            exg0 = exv0[pl.ds(g * 16, 16)]
            exg1 = exv1[pl.ds(g * 16, 16)]
            for j in range(16):
                r = g * 16 + j
                s0 = exg0[j]
                s1 = exg1[j]
                for u in range(4):
                    sl = pl.ds(u * 16, 16)
                    srows[b][r, sl] = grows[b][r, sl] * s0
                for u in range(4):
                    sl = pl.ds(64 + u * 16, 16)
                    srows[b][r, sl] = grows[b][r, sl] * s1
            return 0
        lax.fori_loop(0, CE // 16, sgrp, 0)
        for g in range(CE // 16):
            sl = pl.ds(g * 16, 16)
            dsca[b][sl] = dstv[b][sl]
        pltpu.async_copy(srows[b], agg_sh.at[dsca[b]], sems[b], add=True)

    def hpass(hp, _):
        def zcp(j, _):
            pltpu.sync_copy(zbuf, agg_sh.at[pl.ds(s * RPT + j * 16, 16)])
            return 0
        lax.fori_loop(0, RPT // 16, zcp, 0)
        plsc.subcore_barrier()

        load_idx(0, 0, hp)

        def pair(t, _):
            ch = 2 * t
            load_idx(ch + 1, 1, hp)
            process(ch, 0, hp, t > 0)
            load_idx(ch + 2, 0, hp)
            process(ch + 1, 1, hp, t > 0)
            return 0
        lax.fori_loop(0, NCH // 2, pair, 0)
        process(NCH - 1, 0, hp, True)

        pltpu.make_async_copy(v_hbm.at[pl.ds(0, CE)], srows[0], sems[0]).wait()
        pltpu.make_async_copy(v_hbm.at[pl.ds(0, CE)], srows[1], sems[1]).wait()
        plsc.subcore_barrier()
        pltpu.sync_copy(agg_sh.at[pl.ds(s * RPT, RPT)],
                        agg_hbm.at[c, pl.ds(s * RPT, RPT), pl.ds(hp * 128, 128)])
        plsc.subcore_barrier()
        return 0
    lax.fori_loop(0, H1 // 2, hpass, 0)


def _aggr1(v4, src, dst, exh):
    f = pl.kernel(
        _aggr1_body,
        mesh=plsc.VectorSubcoreMesh(core_axis_name="c", subcore_axis_name="s"),
        compiler_params=pltpu.CompilerParams(use_tc_tiling_on_sc=False, needs_layout_passes=False),
        out_type=[jax.ShapeDtypeStruct((NC, NP, HID), jnp.float32)],
        scratch_types=[
            pltpu.VMEM((CE, 128), jnp.float32),
            pltpu.VMEM((CE, 128), jnp.float32),
            pltpu.VMEM((CE, 128), jnp.float32),
            pltpu.VMEM((CE, 128), jnp.float32),
            pltpu.VMEM((CE,), jnp.int32),
            pltpu.VMEM((CE,), jnp.int32),
            pltpu.VMEM((CE,), jnp.int32),
            pltpu.VMEM((CE,), jnp.int32),
            pltpu.VMEM((CE,), jnp.int32),
            pltpu.VMEM((CE,), jnp.int32),
            pltpu.VMEM((CE,), jnp.int32),
            pltpu.VMEM((CE,), jnp.int32),
            pltpu.VMEM((CE,), jnp.float32),
            pltpu.VMEM((CE,), jnp.float32),
            pltpu.VMEM((16, 128), jnp.float32),
            pltpu.VMEM_SHARED((NP, 128), jnp.float32),
            pltpu.SemaphoreType.DMA,
            pltpu.SemaphoreType.DMA,
            pltpu.SemaphoreType.DMA,
            pltpu.SemaphoreType.DMA,
        ],
    )
    return f(v4, src, dst, exh)[0]


def _mid_body(ap_ref, dp_ref, wa_ref, wk2_ref, wq2_ref, wv2_ref,
              ar2_ref, mr2_ref, pr2_ref, kqv_out):
    agg = ap_ref[0] + ap_ref[1]
    den = dp_ref[0] + dp_ref[1]
    den8 = den[:, :H1] + 1e-16
    div = jnp.repeat(den8, D1, axis=1)
    h1 = jax.nn.gelu(agg / div)
    h1 = jnp.dot(h1, wa_ref[...], preferred_element_type=jnp.float32)
    wk2e = jnp.dot(wk2_ref[...], ar2_ref[...], preferred_element_type=jnp.float32)
    wv2e = jnp.dot(wv2_ref[...], mr2_ref[...], preferred_element_type=jnp.float32)
    k2 = jnp.dot(h1, wk2e, preferred_element_type=jnp.float32)
    q2 = jnp.dot(h1, wq2_ref[...], preferred_element_type=jnp.float32)
    q2 = q2 * (pr2_ref[0, 0] / jnp.sqrt(jnp.float32(OUT)))
    v2 = jnp.dot(h1, wv2e, preferred_element_type=jnp.float32)
    kqv_out[...] = jnp.concatenate(
        [k2, q2, v2, jnp.zeros((BN, 10), jnp.float32)], axis=1)


def _mid(agg1, den1, wa, wk2, wq2, wv2, ar2, mr2, pr2):
    f = pl.pallas_call(
        _mid_body,
        grid=(N // BN,),
        in_specs=[
            pl.BlockSpec((NC, BN, HID), lambda i: (0, i, 0)),
            pl.BlockSpec((NC, BN, 16), lambda i: (0, i, 0)),
            pl.BlockSpec((HID, HID), lambda i: (0, 0)),
            pl.BlockSpec((HID, OUT), lambda i: (0, 0)),
            pl.BlockSpec((HID, OUT), lambda i: (0, 0)),
            pl.BlockSpec((HID, OUT), lambda i: (0, 0)),
            pl.BlockSpec((OUT, OUT), lambda i: (0, 0)),
            pl.BlockSpec((OUT, OUT), lambda i: (0, 0)),
            pl.BlockSpec((1, 1), lambda i: (0, 0)),
        ],
        out_specs=[pl.BlockSpec((BN, 16), lambda i: (i, 0))],
        out_shape=[jax.ShapeDtypeStruct((N, 16), jnp.float32)],
    )
    return f(agg1, den1, wa, wk2, wq2, wv2, ar2, mr2, pr2)[0]


def _l2_body(kqv_hbm, src_hbm, dst_hbm, out_hbm,
             srows, drows, srcv, dstv, outrows, zbuf, out_sh, sem1, sem2):
    c = lax.axis_index("c")
    s = lax.axis_index("s")
    wid = c * NS + s
    e0 = wid * EW

    def zrow(i, _):
        outrows[i, :] = jnp.zeros((16,), jnp.float32)
        return 0
    lax.fori_loop(0, CE, zrow, 0)

    def zline(i, _):
        zbuf[i, :] = jnp.zeros((16,), jnp.float32)
        return 0
    lax.fori_loop(0, 128, zline, 0)
    for j in range(5):
        pltpu.sync_copy(zbuf, out_sh.at[pl.ds(s * RPT + j * 128, 128)])
    plsc.subcore_barrier()

    def chunk(i, _):
        off = e0 + i * CE
        pltpu.sync_copy(src_hbm.at[pl.ds(off, CE)], srcv)
        pltpu.sync_copy(dst_hbm.at[pl.ds(off, CE)], dstv)
        cp1 = pltpu.async_copy(kqv_hbm.at[srcv], srows, sem1)
        cp2 = pltpu.async_copy(kqv_hbm.at[dstv], drows, sem2)
        cp1.wait()
        cp2.wait()
        for g in range(CE // 16):
            rows = g * 16 + _iota16()
            k0 = plsc.load_gather(srows, [rows, jnp.full((16,), 0, jnp.int32)])
            k1 = plsc.load_gather(srows, [rows, jnp.full((16,), 1, jnp.int32)])
            q0 = plsc.load_gather(drows, [rows, jnp.full((16,), 2, jnp.int32)])
            q1 = plsc.load_gather(drows, [rows, jnp.full((16,), 3, jnp.int32)])
            v0 = plsc.load_gather(srows, [rows, jnp.full((16,), 4, jnp.int32)])
            v1 = plsc.load_gather(srows, [rows, jnp.full((16,), 5, jnp.int32)])
            ex = jnp.exp(k0 * q0 + k1 * q1)
            plsc.store_scatter(outrows, [rows, jnp.full((16,), 0, jnp.int32)], v0 * ex)
            plsc.store_scatter(outrows, [rows, jnp.full((16,), 1, jnp.int32)], v1 * ex)
            plsc.store_scatter(outrows, [rows, jnp.full((16,), 2, jnp.int32)], ex)
        pltpu.sync_copy(outrows, out_sh.at[dstv], add=True)
        return 0
    lax.fori_loop(0, NCH, chunk, 0)

    plsc.subcore_barrier()
    pltpu.sync_copy(out_sh.at[pl.ds(s * RPT, RPT)],
                    out_hbm.at[c, pl.ds(s * RPT, RPT)])


def _l2(kqv2, src, dst):
    f = pl.kernel(
        _l2_body,
        mesh=plsc.VectorSubcoreMesh(core_axis_name="c", subcore_axis_name="s"),
        compiler_params=pltpu.CompilerParams(use_tc_tiling_on_sc=False, needs_layout_passes=False),
        out_type=[jax.ShapeDtypeStruct((NC, NP, 16), jnp.float32)],
        scratch_types=[
            pltpu.VMEM((CE, 16), jnp.float32),
            pltpu.VMEM((CE, 16), jnp.float32),
            pltpu.VMEM((CE,), jnp.int32),
            pltpu.VMEM((CE,), jnp.int32),
            pltpu.VMEM((CE, 16), jnp.float32),
            pltpu.VMEM((128, 16), jnp.float32),
            pltpu.VMEM_SHARED((NP, 16), jnp.float32),
            pltpu.SemaphoreType.DMA,
            pltpu.SemaphoreType.DMA,
        ],
    )
    return f(kqv2, src, dst)[0]


def _out_body(p2_ref, wa2_ref, o_ref):
    t = p2_ref[0] + p2_ref[1]
    num = t[:N, 0:2]
    den = t[:N, 2:3] + 1e-16
    o_ref[...] = jnp.dot(jax.nn.gelu(num / den), wa2_ref[...],
                         preferred_element_type=jnp.float32)


def _out(p2, wa2):
    f = pl.pallas_call(
        _out_body,
        in_specs=[
            pl.BlockSpec((NC, NP, 16), lambda: (0, 0, 0)),
            pl.BlockSpec((OUT, OUT), lambda: (0, 0)),
        ],
        out_specs=[pl.BlockSpec((N, OUT), lambda: (0, 0))],
        out_shape=[jax.ShapeDtypeStruct((N, OUT), jnp.float32)],
    )
    return f(p2, wa2)[0]


def kernel(x, edge_index, Wk1, Wq1, Wv1, a_rel1, m_rel1, pri1, Wa1,
           Wk2, Wq2, Wv2, a_rel2, m_rel2, pri2, Wa2):
    src = edge_index[0].astype(jnp.int32)
    dst = edge_index[1].astype(jnp.int32)
    qscale = (jnp.repeat(pri1, D1) / jnp.sqrt(jnp.float32(D1))).reshape(1, HID)
    k1, q1, v1 = _proj1(x, Wk1, Wq1, Wv1, a_rel1, m_rel1, qscale)
    exh, den1 = _attn1(k1, q1, src, dst)
    agg1 = _aggr1(v1.reshape(N * 4, 128), src, dst, exh)
    kqv2 = _mid(agg1, den1, Wa1, Wk2, Wq2, Wv2,
                a_rel2.reshape(OUT, OUT), m_rel2.reshape(OUT, OUT),
                pri2.reshape(1, 1))
    p2 = _l2(kqv2, src, dst)
    return _out(p2, Wa2)
